# baseline (speedup 1.0000x reference)
import functools

import jax
import jax.numpy as jnp
from jax import lax
from jax.experimental import pallas as pl
from jax.experimental.pallas import tpu as pltpu
from jax.experimental.pallas import tpu_sc as plsc

B = 320000
D = 128
NC = 2
NS = 16
NW = NC * NS
B_PER_W = B // NW
CHUNK = 160
L = 16
NG = B_PER_W // L
SEG = 5
NGS = NG // SEG
U = 5
TRASH = B_PER_W + (SEG + 1) * CHUNK
FLAT = TRASH + L
BIGLOC = 1 << 20

_mesh = plsc.VectorSubcoreMesh(core_axis_name="c", subcore_axis_name="s")


@functools.partial(
    pl.kernel,
    mesh=_mesh,
    out_type=jax.ShapeDtypeStruct((B, D), jnp.float32),
    scratch_types=[
        pltpu.VMEM((B_PER_W,), jnp.int32),
        pltpu.VMEM((FLAT,), jnp.int32),
        pltpu.VMEM((FLAT,), jnp.int32),
        pltpu.VMEM((CHUNK, D), jnp.float32),
        pltpu.VMEM((CHUNK, D), jnp.float32),
        pltpu.VMEM((B_PER_W,), jnp.int32),
        pltpu.VMEM((CHUNK,), jnp.int32),
        pltpu.VMEM((CHUNK,), jnp.int32),
        pltpu.SMEM((NG,), jnp.int32),
        pltpu.SMEM((NG,), jnp.int32),
        pltpu.VMEM_SHARED((2, D), jnp.float32),
        pltpu.SemaphoreType.DMA,
        pltpu.SemaphoreType.DMA,
    ],
    compiler_params=pltpu.CompilerParams(needs_layout_passes=False),
)
def _lookup(x_hbm, lin_hbm, out_hbm, idx_v, flat0, flat1, rows0, rows1,
            qbuf, fidx0, fidx1, cnt_sm, base_sm, table_sh, fill_sem,
            sc_sem):
    sid = lax.axis_index("s")
    wid = sid * NC + lax.axis_index("c")
    base = wid * B_PER_W

    @pl.when(sid == 0)
    def _():
        pltpu.sync_copy(lin_hbm, table_sh)

    pltpu.sync_copy(x_hbm.at[pl.ds(base, B_PER_W)], idx_v)
    plsc.subcore_barrier()

    zeros = jnp.zeros((L,), jnp.int32)
    ones = jnp.ones((L,), jnp.int32)
    for k in range(CHUNK // L):
        fidx0[pl.ds(k * L, L)] = zeros
        fidx1[pl.ds(k * L, L)] = ones
    fill0 = pltpu.make_async_copy(table_sh.at[fidx0], rows0, fill_sem)
    fill1 = pltpu.make_async_copy(table_sh.at[fidx1], rows1, fill_sem)
    fill0.start()
    fill1.start()

    iota = lax.iota(jnp.int32, L)
    bigloc = jnp.full((L,), jnp.int32(BIGLOC))
    trashv = jnp.full((L,), jnp.int32(TRASH))
    lane0 = jnp.zeros((L,), jnp.int32)

    def _bcast_lane0(v):
        return lax.gather(
            v, lane0[:, None],
            lax.GatherDimensionNumbers(
                offset_dims=(), collapsed_slice_dims=(0,),
                start_index_map=(0,)),
            slice_sizes=(1,),
            mode=lax.GatherScatterMode.PROMISE_IN_BOUNDS)

    def fire0(k, carry):
        pltpu.make_async_copy(
            rows0, out_hbm.at[flat0.at[pl.ds(k * CHUNK, CHUNK)]], sc_sem
        ).start()
        return carry

    def fire1(k, carry):
        pltpu.make_async_copy(
            rows1, out_hbm.at[flat1.at[pl.ds(k * CHUNK, CHUNK)]], sc_sem
        ).start()
        return carry

    filled = False
    c0r = jnp.int32(0)
    c1r = jnp.int32(0)
    totch = jnp.int32(0)

    for s in range(SEG):
        g0 = s * NGS

        def phase1(k, carry):
            for u in range(U):
                g = g0 + k * U + u
                xv = idx_v[pl.ds(g * L, L)]
                q = plsc.cumsum(xv)
                qbuf[pl.ds(g * L, L)] = q
                cnt_sm[g] = jnp.max(q)
            return carry

        lax.fori_loop(0, NGS // U, phase1, 0)

        def phase2(k, c):
            for u in range(U):
                g = g0 + k * U + u
                base_sm[g] = c
                c = c + cnt_sm[g]
            return c

        c1_end = lax.fori_loop(0, NGS // U, phase2, c1r)
        c0_end = c0r + NGS * L - (c1_end - c1r)

        def phase3(k, carry):
            for u in range(U):
                g = g0 + k * U + u
                b1s = base_sm[g]
                b1 = jnp.full((L,), b1s)
                b0 = jnp.full((L,), c0r + (g - g0) * L - (b1s - c1r))
                q = qbuf[pl.ds(g * L, L)]
                xv = idx_v[pl.ds(g * L, L)]
                m0 = xv == 0
                rowid = base + g * L + iota
                pos0 = jnp.minimum(
                    b0 + jnp.where(m0, iota - q, bigloc), trashv)
                pos1 = jnp.minimum(
                    b1 + jnp.where(m0, bigloc, q - 1), trashv)
                plsc.store_scatter(flat0, [pos0], rowid)
                plsc.store_scatter(flat1, [pos1], rowid)
            return carry

        lax.fori_loop(0, NGS // U, phase3, 0)

        pad0 = _bcast_lane0(flat0[pl.ds(0, L)])
        pad1 = _bcast_lane0(flat1[pl.ds(0, L)])
        for k in range(CHUNK // L):
            plsc.store_scatter(flat0, [c0_end + k * L + iota], pad0)
            plsc.store_scatter(flat1, [c1_end + k * L + iota], pad1)

        k0_lo = c0r // CHUNK
        k0_hi = (c0_end + CHUNK - 1) // CHUNK
        k1_lo = c1r // CHUNK
        k1_hi = (c1_end + CHUNK - 1) // CHUNK

        if not filled:
            fill0.wait()
            fill1.wait()
            filled = True

        lax.fori_loop(k0_lo, k0_hi, fire0, 0)
        lax.fori_loop(k1_lo, k1_hi, fire1, 0)
        totch = totch + (k0_hi - k0_lo) + (k1_hi - k1_lo)
        c0r = k0_hi * CHUNK
        c1r = k1_hi * CHUNK

    def drain(k, carry):
        pltpu.make_async_copy(
            rows0, out_hbm.at[flat0.at[pl.ds(0, CHUNK)]], sc_sem
        ).wait()
        return carry

    lax.fori_loop(0, totch, drain, 0)


def kernel(x, lin):
    out = _lookup(x.astype(jnp.int32), lin)
    return out.reshape(B, 1, D)


# speedup vs baseline: 1.6074x; 1.6074x over previous
import functools

import jax
import jax.numpy as jnp
from jax import lax
from jax.experimental import pallas as pl
from jax.experimental.pallas import tpu as pltpu
from jax.experimental.pallas import tpu_sc as plsc

B = 320000
D = 128
NC = 2
NS = 16
NW = NC * NS
B_PER_W = B // NW
CHUNK = 320
L = 16
NG = B_PER_W // L
SLICE = 5
NGS = NG // SLICE
U = 5
TRASH = B_PER_W + CHUNK
FLAT = TRASH + L
BIGLOC = 1 << 20

_mesh = plsc.VectorSubcoreMesh(core_axis_name="c", subcore_axis_name="s")


@functools.partial(
    pl.kernel,
    mesh=_mesh,
    out_type=jax.ShapeDtypeStruct((B, D), jnp.float32),
    scratch_types=[
        pltpu.VMEM((B_PER_W,), jnp.int32),
        pltpu.VMEM((FLAT,), jnp.int32),
        pltpu.VMEM((FLAT,), jnp.int32),
        pltpu.VMEM((CHUNK, D), jnp.float32),
        pltpu.VMEM((CHUNK, D), jnp.float32),
        pltpu.VMEM((B_PER_W,), jnp.int32),
        pltpu.VMEM((CHUNK,), jnp.int32),
        pltpu.VMEM((CHUNK,), jnp.int32),
        pltpu.SMEM((NG,), jnp.int32),
        pltpu.SMEM((NG,), jnp.int32),
        pltpu.VMEM_SHARED((2, D), jnp.float32),
        pltpu.SemaphoreType.DMA,
        pltpu.SemaphoreType.DMA,
    ],
    compiler_params=pltpu.CompilerParams(needs_layout_passes=False),
)
def _lookup(x_hbm, lin_hbm, out_hbm, idx_v, flat0, flat1, rows0, rows1,
            qbuf, fidx0, fidx1, cnt_sm, base_sm, table_sh, fill_sem,
            sc_sem):
    sid = lax.axis_index("s")
    wid = sid * NC + lax.axis_index("c")
    base = wid * B_PER_W

    @pl.when(sid == 0)
    def _():
        pltpu.sync_copy(lin_hbm, table_sh)

    pltpu.sync_copy(x_hbm.at[pl.ds(base, B_PER_W)], idx_v)
    plsc.subcore_barrier()

    zeros = jnp.zeros((L,), jnp.int32)
    ones = jnp.ones((L,), jnp.int32)
    for k in range(CHUNK // L):
        fidx0[pl.ds(k * L, L)] = zeros
        fidx1[pl.ds(k * L, L)] = ones
    fill0 = pltpu.make_async_copy(table_sh.at[fidx0], rows0, fill_sem)
    fill1 = pltpu.make_async_copy(table_sh.at[fidx1], rows1, fill_sem)
    fill0.start()
    fill1.start()

    iota = lax.iota(jnp.int32, L)
    bigloc = jnp.full((L,), jnp.int32(BIGLOC))
    trashv = jnp.full((L,), jnp.int32(TRASH))
    lane0 = jnp.zeros((L,), jnp.int32)

    def phase1(k, carry):
        for u in range(U):
            g = k * U + u
            xv = idx_v[pl.ds(g * L, L)]
            q = plsc.cumsum(xv)
            qbuf[pl.ds(g * L, L)] = q
            cnt_sm[g] = jnp.max(q)
        return carry

    lax.fori_loop(0, NG // U, phase1, 0)

    def phase2(k, c):
        for u in range(U):
            g = k * U + u
            base_sm[g] = c
            c = c + cnt_sm[g]
        return c

    c1 = lax.fori_loop(0, NG // U, phase2, jnp.int32(0))
    c0 = B_PER_W - c1

    fill0.wait()
    fill1.wait()

    def fire0(k, carry):
        pltpu.make_async_copy(
            rows0, out_hbm.at[flat0.at[pl.ds(k * CHUNK, CHUNK)]], sc_sem
        ).start()
        return carry

    def fire1(k, carry):
        pltpu.make_async_copy(
            rows1, out_hbm.at[flat1.at[pl.ds(k * CHUNK, CHUNK)]], sc_sem
        ).start()
        return carry

    k0_fired = jnp.int32(0)
    k1_fired = jnp.int32(0)
    for s in range(SLICE):
        def phase3(k, carry):
            for u in range(U):
                g = s * NGS + k * U + u
                b1s = base_sm[g]
                b1 = jnp.full((L,), b1s)
                b0 = jnp.full((L,), g * L - b1s)
                q = qbuf[pl.ds(g * L, L)]
                xv = idx_v[pl.ds(g * L, L)]
                m0 = xv == 0
                rowid = base + g * L + iota
                pos0 = jnp.minimum(
                    b0 + jnp.where(m0, iota - q, bigloc), trashv)
                pos1 = jnp.minimum(
                    b1 + jnp.where(m0, bigloc, q - 1), trashv)
                plsc.store_scatter(flat0, [pos0], rowid)
                plsc.store_scatter(flat1, [pos1], rowid)
            return carry

        lax.fori_loop(0, NGS // U, phase3, 0)

        if s < SLICE - 1:
            gn = (s + 1) * NGS
            b1n = base_sm[gn]
            b0n = gn * L - b1n
            k0_done = b0n // CHUNK
            k1_done = b1n // CHUNK
            lax.fori_loop(k0_fired, k0_done, fire0, 0)
            lax.fori_loop(k1_fired, k1_done, fire1, 0)
            k0_fired = k0_done
            k1_fired = k1_done

    def _bcast_lane0(v):
        return lax.gather(
            v, lane0[:, None],
            lax.GatherDimensionNumbers(
                offset_dims=(), collapsed_slice_dims=(0,),
                start_index_map=(0,)),
            slice_sizes=(1,),
            mode=lax.GatherScatterMode.PROMISE_IN_BOUNDS)

    pad0 = _bcast_lane0(flat0[pl.ds(0, L)])
    pad1 = _bcast_lane0(flat1[pl.ds(0, L)])
    for k in range(CHUNK // L):
        plsc.store_scatter(flat0, [c0 + k * L + iota], pad0)
        plsc.store_scatter(flat1, [c1 + k * L + iota], pad1)

    nch0 = (c0 + CHUNK - 1) // CHUNK
    nch1 = (c1 + CHUNK - 1) // CHUNK
    lax.fori_loop(k0_fired, nch0, fire0, 0)
    lax.fori_loop(k1_fired, nch1, fire1, 0)

    def drain(k, carry):
        pltpu.make_async_copy(
            rows0, out_hbm.at[flat0.at[pl.ds(0, CHUNK)]], sc_sem
        ).wait()
        return carry

    lax.fori_loop(0, nch0 + nch1, drain, 0)


def kernel(x, lin):
    out = _lookup(x.astype(jnp.int32), lin)
    return out.reshape(B, 1, D)
